# baseline (speedup 1.0000x reference)
import functools

import jax
import jax.numpy as jnp
from jax import lax
from jax.experimental import pallas as pl
from jax.experimental.pallas import tpu as pltpu
from jax.experimental.pallas import tpu_sc as plsc

A = 16
S = 65536
L = 16
C = 64
Q = S // C
FCH = A * A * C
TCH = A * C
N_ITER = 10
MASK_STD = 0.8


def _sc_info():
    try:
        info = plsc.get_sparse_core_info()
        return info.num_cores, info.num_subcores
    except Exception:
        return 2, 16


def _worker_id():
    return lax.axis_index("c") * 16 + lax.axis_index("s")


def _tree16(p):
    def blk(q):
        s = [q[j] + q[j + 4] for j in range(4)]
        s = [s[j] + s[j + 2] for j in range(2)]
        return s[0] + s[1]
    return blk(p[:8]) + blk(p[8:])


def _iter1_body(nw, want_aux, fT_hbm, tp_hbm, wr_hbm, *rest):
    if want_aux:
        (out_hbm, g_hbm, oh_hbm, wr_v, f_v, tp_v, o_v, g_v, oh_v,
         sem_f0, sem_f1, sem_tp0, sem_tp1) = rest
    else:
        (out_hbm, wr_v, f_v, tp_v, o_v,
         sem_f0, sem_f1, sem_tp0, sem_tp1) = rest
    sem_f = (sem_f0, sem_f1)
    sem_tp = (sem_tp0, sem_tp1)
    cpt = Q // nw
    wid = _worker_id()
    q0 = wid * cpt

    for s in range(2):
        pltpu.async_copy(fT_hbm.at[q0 + s], f_v.at[pl.ds(s * FCH, FCH)],
                         sem_f[s])
        pltpu.async_copy(tp_hbm.at[q0 + s], tp_v.at[pl.ds(s * TCH, TCH)],
                         sem_tp[s])
    pltpu.sync_copy(wr_hbm, wr_v)

    def phase(t, s):
        q = q0 + t
        pltpu.make_async_copy(fT_hbm.at[q], f_v.at[pl.ds(s * FCH, FCH)],
                              sem_f[s]).wait()
        pltpu.make_async_copy(tp_hbm.at[q], tp_v.at[pl.ds(s * TCH, TCH)],
                              sem_tp[s]).wait()

        def batch(b, _):
            base = b * L
            tpv = [tp_v[pl.ds(s * TCH + j * C + base, L)] for j in range(A)]
            m = None
            bi = None
            for k in range(A):
                prods = []
                for j in range(A):
                    idx = f_v[pl.ds(s * FCH + (j * A + k) * C + base, L)]
                    g = plsc.load_gather(wr_v, [idx])
                    prods.append(tpv[j] * g)
                tk = _tree16(prods)
                if k == 0:
                    m = tk
                    if want_aux:
                        bi = jnp.zeros((L,), jnp.int32)
                else:
                    if want_aux:
                        bi = jnp.where(tk > m, jnp.full((L,), k, jnp.int32), bi)
                    m = jnp.maximum(m, tk)
            o_v[pl.ds(t * C + base, L)] = m
            if want_aux:
                lane = lax.iota(jnp.int32, L) + (s * FCH + base)
                for j in range(A):
                    sel = plsc.load_gather(f_v, [bi * C + (j * A * C) + lane])
                    g_v[pl.ds(j * C + base, L)] = sel
                    oh_v[pl.ds(j * C + base, L)] = jnp.where(
                        bi == j, jnp.float32(1.0), jnp.float32(0.0))
            return 0

        for _b in range(C // L):
            batch(_b, 0)
        if want_aux:
            pltpu.sync_copy(g_v, g_hbm.at[q])
            pltpu.sync_copy(oh_v, oh_hbm.at[q])

        @pl.when(t + 2 < cpt)
        def _():
            pltpu.async_copy(fT_hbm.at[q + 2], f_v.at[pl.ds(s * FCH, FCH)],
                             sem_f[s])
            pltpu.async_copy(tp_hbm.at[q + 2], tp_v.at[pl.ds(s * TCH, TCH)],
                             sem_tp[s])

    def pair(i, _):
        phase(2 * i, 0)
        phase(2 * i + 1, 1)
        return 0

    lax.fori_loop(0, cpt // 2, pair, 0)
    pltpu.sync_copy(o_v, out_hbm.at[pl.ds(q0 * C, cpt * C)])


def _iter2_body(nw, g_hbm, tp_hbm, wr_hbm, out_hbm, wr_v, g_v, tp_v, o_v,
                sem_g0, sem_g1, sem_tp0, sem_tp1):
    sem_g = (sem_g0, sem_g1)
    sem_tp = (sem_tp0, sem_tp1)
    cpt = Q // nw
    wid = _worker_id()
    q0 = wid * cpt

    for s in range(2):
        pltpu.async_copy(g_hbm.at[q0 + s], g_v.at[pl.ds(s * TCH, TCH)],
                         sem_g[s])
        pltpu.async_copy(tp_hbm.at[q0 + s], tp_v.at[pl.ds(s * TCH, TCH)],
                         sem_tp[s])
    pltpu.sync_copy(wr_hbm, wr_v)

    def phase(t, s):
        q = q0 + t
        pltpu.make_async_copy(g_hbm.at[q], g_v.at[pl.ds(s * TCH, TCH)],
                              sem_g[s]).wait()
        pltpu.make_async_copy(tp_hbm.at[q], tp_v.at[pl.ds(s * TCH, TCH)],
                              sem_tp[s]).wait()

        def batch(b, _):
            base = b * L
            prods = []
            for j in range(A):
                idx = g_v[pl.ds(s * TCH + j * C + base, L)]
                g = plsc.load_gather(wr_v, [idx])
                prods.append(tp_v[pl.ds(s * TCH + j * C + base, L)] * g)
            o_v[pl.ds(t * C + base, L)] = _tree16(prods)
            return 0

        for _b in range(C // L):
            batch(_b, 0)

        @pl.when(t + 2 < cpt)
        def _():
            pltpu.async_copy(g_hbm.at[q + 2], g_v.at[pl.ds(s * TCH, TCH)],
                             sem_g[s])
            pltpu.async_copy(tp_hbm.at[q + 2], tp_v.at[pl.ds(s * TCH, TCH)],
                             sem_tp[s])

    def pair(i, _):
        phase(2 * i, 0)
        phase(2 * i + 1, 1)
        return 0

    lax.fori_loop(0, cpt // 2, pair, 0)
    pltpu.sync_copy(o_v, out_hbm.at[pl.ds(q0 * C, cpt * C)])


def kernel(policy, F, init_wr):
    nc, ns = _sc_info()
    nw = nc * ns
    mesh = plsc.VectorSubcoreMesh(core_axis_name="c", subcore_axis_name="s")
    cparams = pltpu.CompilerParams(needs_layout_passes=False)
    f32 = jnp.float32

    mx = jnp.max(policy, axis=1, keepdims=True)
    e = jnp.exp(policy - mx)
    es = [e[:, j] for j in range(A)]
    s8 = [es[j] + es[j + 8] for j in range(8)]
    s4 = [s8[j] + s8[j + 4] for j in range(4)]
    s2 = [s4[j] + s4[j + 2] for j in range(2)]
    den = s2[0] + s2[1]
    pd = e / den[:, None]

    tp = pd.reshape(16, 16, 16, 16, A).transpose(1, 0, 3, 2, 4).reshape(S, A)
    tpR = tp.reshape(Q, C, A).transpose(0, 2, 1).reshape(Q, TCH)
    fT = F.reshape(Q, C, A * A).transpose(0, 2, 1).reshape(Q, FCH)

    spt = S // nw
    dma_sems = [pltpu.SemaphoreType.DMA] * 4

    iter1_k = pl.kernel(
        functools.partial(_iter1_body, nw, False),
        out_type=jax.ShapeDtypeStruct((S,), f32),
        mesh=mesh,
        compiler_params=cparams,
        scratch_types=[
            pltpu.VMEM((S,), f32),
            pltpu.VMEM((2 * FCH,), jnp.int32),
            pltpu.VMEM((2 * TCH,), f32),
            pltpu.VMEM((spt,), f32),
        ] + dma_sems,
    )

    final1_k = pl.kernel(
        functools.partial(_iter1_body, nw, True),
        out_type=(
            jax.ShapeDtypeStruct((S,), f32),
            jax.ShapeDtypeStruct((Q, TCH), jnp.int32),
            jax.ShapeDtypeStruct((Q, TCH), f32),
        ),
        mesh=mesh,
        compiler_params=cparams,
        scratch_types=[
            pltpu.VMEM((S,), f32),
            pltpu.VMEM((2 * FCH,), jnp.int32),
            pltpu.VMEM((2 * TCH,), f32),
            pltpu.VMEM((spt,), f32),
            pltpu.VMEM((TCH,), jnp.int32),
            pltpu.VMEM((TCH,), f32),
        ] + dma_sems,
    )

    iter2_k = pl.kernel(
        functools.partial(_iter2_body, nw),
        out_type=jax.ShapeDtypeStruct((S,), f32),
        mesh=mesh,
        compiler_params=cparams,
        scratch_types=[
            pltpu.VMEM((S,), f32),
            pltpu.VMEM((2 * TCH,), jnp.int32),
            pltpu.VMEM((2 * TCH,), f32),
            pltpu.VMEM((spt,), f32),
        ] + dma_sems,
    )

    twr = init_wr
    for _ in range(N_ITER - 1):
        twr = iter1_k(fT, tpR, twr)
    twr, gT, ohT = final1_k(fT, tpR, twr)

    wr2 = init_wr
    for _ in range(N_ITER):
        wr2 = iter2_k(gT, tpR, wr2)

    twr4 = twr.reshape(16, 16, 16, 16)
    v = twr4[1, 1, 0, 0]
    ub = 1.0 - (1.0 - v) / (v + 1e-9)

    mask_key = jax.random.key(42)
    mask = (jax.random.uniform(mask_key, twr4.shape) < MASK_STD).astype(f32)
    loss = jnp.sum(mask * twr4)

    opt_policy_dist = (ohT.reshape(Q, A, C).transpose(0, 2, 1)
                       .reshape(S, A).astype(policy.dtype))

    wr4 = wr2.reshape(16, 16, 16, 16)
    v2 = wr4[1, 1, 0, 0]
    lb = 1.0 - (1.0 - v2) / (v2 + 1e-9)

    return (ub, lb, opt_policy_dist, twr4, loss)


# speedup vs baseline: 1.8414x; 1.8414x over previous
import functools

import jax
import jax.numpy as jnp
from jax import lax
from jax.experimental import pallas as pl
from jax.experimental.pallas import tpu as pltpu
from jax.experimental.pallas import tpu_sc as plsc

A = 16
S = 65536
L = 16
C = 64
Q = S // C
FCH = A * A * C
TCH = A * C
N_ITER = 10
MASK_STD = 0.8


def _sc_info():
    try:
        info = plsc.get_sparse_core_info()
        return info.num_cores, info.num_subcores
    except Exception:
        return 2, 16


def _worker_id():
    return lax.axis_index("c") * 16 + lax.axis_index("s")


def _tree16(p):
    def blk(q):
        s = [q[j] + q[j + 4] for j in range(4)]
        s = [s[j] + s[j + 2] for j in range(2)]
        return s[0] + s[1]
    return blk(p[:8]) + blk(p[8:])


def _iter1_body(nw, want_aux, fT_hbm, tp_hbm, wr_hbm, *rest):
    if want_aux:
        (out_hbm, g_hbm, oh_hbm, wr_v, f_v, tp_v, o_v, g_v, oh_v,
         sem_f0, sem_f1, sem_tp0, sem_tp1) = rest
    else:
        (out_hbm, wr_v, f_v, tp_v, o_v,
         sem_f0, sem_f1, sem_tp0, sem_tp1) = rest
    sem_f = (sem_f0, sem_f1)
    sem_tp = (sem_tp0, sem_tp1)
    cpt = Q // nw
    wid = _worker_id()
    q0 = wid * cpt

    for s in range(2):
        pltpu.async_copy(fT_hbm.at[q0 + s], f_v.at[pl.ds(s * FCH, FCH)],
                         sem_f[s])
        pltpu.async_copy(tp_hbm.at[q0 + s], tp_v.at[pl.ds(s * TCH, TCH)],
                         sem_tp[s])
    pltpu.sync_copy(wr_hbm, wr_v)

    def phase(t, s):
        q = q0 + t
        pltpu.make_async_copy(fT_hbm.at[q], f_v.at[pl.ds(s * FCH, FCH)],
                              sem_f[s]).wait()
        pltpu.make_async_copy(tp_hbm.at[q], tp_v.at[pl.ds(s * TCH, TCH)],
                              sem_tp[s]).wait()

        def batch(b, _):
            base = b * L
            tpv = [tp_v[pl.ds(s * TCH + j * C + base, L)] for j in range(A)]
            m = None
            bi = None
            for k in range(A):
                prods = []
                for j in range(A):
                    idx = f_v[pl.ds(s * FCH + (j * A + k) * C + base, L)]
                    g = plsc.load_gather(wr_v, [idx])
                    prods.append(tpv[j] * g)
                tk = _tree16(prods)
                if k == 0:
                    m = tk
                    if want_aux:
                        bi = jnp.zeros((L,), jnp.int32)
                else:
                    if want_aux:
                        bi = jnp.where(tk > m, jnp.full((L,), k, jnp.int32), bi)
                    m = jnp.maximum(m, tk)
            o_v[pl.ds(t * C + base, L)] = m
            if want_aux:
                lane = lax.iota(jnp.int32, L) + (s * FCH + base)
                for j in range(A):
                    sel = plsc.load_gather(f_v, [bi * C + (j * A * C) + lane])
                    g_v[pl.ds(j * C + base, L)] = sel
                    oh_v[pl.ds(j * C + base, L)] = jnp.where(
                        bi == j, jnp.float32(1.0), jnp.float32(0.0))
            return 0

        lax.fori_loop(0, C // L, batch, 0)
        if want_aux:
            pltpu.sync_copy(g_v, g_hbm.at[q])
            pltpu.sync_copy(oh_v, oh_hbm.at[q])

        @pl.when(t + 2 < cpt)
        def _():
            pltpu.async_copy(fT_hbm.at[q + 2], f_v.at[pl.ds(s * FCH, FCH)],
                             sem_f[s])
            pltpu.async_copy(tp_hbm.at[q + 2], tp_v.at[pl.ds(s * TCH, TCH)],
                             sem_tp[s])

    def pair(i, _):
        phase(2 * i, 0)
        phase(2 * i + 1, 1)
        return 0

    lax.fori_loop(0, cpt // 2, pair, 0)
    pltpu.sync_copy(o_v, out_hbm.at[pl.ds(q0 * C, cpt * C)])


def _loop2_body(ns, g_hbm, tp_hbm, wr0_hbm, out_hbm,
                wr_v, g_v, tp_v, o_v, wr_sh,
                sem_g0, sem_g1, sem_tp0, sem_tp1):
    sem_g = (sem_g0, sem_g1)
    sem_tp = (sem_tp0, sem_tp1)
    MC = 4 * TCH
    cpt = Q // (4 * ns)
    spt = cpt * 4 * C
    tid = lax.axis_index("s")
    q0 = tid * cpt

    pltpu.sync_copy(wr0_hbm, wr_v)

    def iteration(it, _):
        for s in range(2):
            pltpu.async_copy(g_hbm.at[q0 + s], g_v.at[pl.ds(s * MC, MC)],
                             sem_g[s])
            pltpu.async_copy(tp_hbm.at[q0 + s], tp_v.at[pl.ds(s * MC, MC)],
                             sem_tp[s])

        def phase(t, s):
            q = q0 + t
            pltpu.make_async_copy(g_hbm.at[q], g_v.at[pl.ds(s * MC, MC)],
                                  sem_g[s]).wait()
            pltpu.make_async_copy(tp_hbm.at[q], tp_v.at[pl.ds(s * MC, MC)],
                                  sem_tp[s]).wait()
            for cc in range(4):
                def batch(b, _):
                    base = b * L
                    off = s * MC + cc * TCH
                    prods = []
                    for j in range(A):
                        idx = g_v[pl.ds(off + j * C + base, L)]
                        g = plsc.load_gather(wr_v, [idx])
                        prods.append(tp_v[pl.ds(off + j * C + base, L)] * g)
                    o_v[pl.ds((t * 4 + cc) * C + base, L)] = _tree16(prods)
                    return 0

                lax.fori_loop(0, C // L, batch, 0)

            @pl.when(t + 2 < cpt)
            def _():
                pltpu.async_copy(g_hbm.at[q + 2], g_v.at[pl.ds(s * MC, MC)],
                                 sem_g[s])
                pltpu.async_copy(tp_hbm.at[q + 2], tp_v.at[pl.ds(s * MC, MC)],
                                 sem_tp[s])

        def pair(i, _):
            phase(2 * i, 0)
            phase(2 * i + 1, 1)
            return 0

        lax.fori_loop(0, cpt // 2, pair, 0)

        pltpu.sync_copy(o_v, wr_sh.at[pl.ds(tid * spt, spt)])
        plsc.subcore_barrier()
        pltpu.sync_copy(wr_sh, wr_v)
        plsc.subcore_barrier()
        return 0

    lax.fori_loop(0, N_ITER, iteration, 0)

    @pl.when(lax.axis_index("c") == 0)
    def _():
        pltpu.sync_copy(o_v, out_hbm.at[pl.ds(q0 * 4 * C, spt)])


def kernel(policy, F, init_wr):
    nc, ns = _sc_info()
    nw = nc * ns
    mesh = plsc.VectorSubcoreMesh(core_axis_name="c", subcore_axis_name="s")
    cparams = pltpu.CompilerParams(needs_layout_passes=False)
    f32 = jnp.float32

    mx = jnp.max(policy, axis=1, keepdims=True)
    e = jnp.exp(policy - mx)
    es = [e[:, j] for j in range(A)]
    s8 = [es[j] + es[j + 8] for j in range(8)]
    s4 = [s8[j] + s8[j + 4] for j in range(4)]
    s2 = [s4[j] + s4[j + 2] for j in range(2)]
    den = s2[0] + s2[1]
    pd = e / den[:, None]

    tp = pd.reshape(16, 16, 16, 16, A).transpose(1, 0, 3, 2, 4).reshape(S, A)
    tpR = tp.reshape(Q, C, A).transpose(0, 2, 1).reshape(Q, TCH)
    fT = F.reshape(Q, C, A * A).transpose(0, 2, 1).reshape(Q, FCH)

    spt = S // nw
    dma_sems = [pltpu.SemaphoreType.DMA] * 4

    iter1_k = pl.kernel(
        functools.partial(_iter1_body, nw, False),
        out_type=jax.ShapeDtypeStruct((S,), f32),
        mesh=mesh,
        compiler_params=cparams,
        scratch_types=[
            pltpu.VMEM((S,), f32),
            pltpu.VMEM((2 * FCH,), jnp.int32),
            pltpu.VMEM((2 * TCH,), f32),
            pltpu.VMEM((spt,), f32),
        ] + dma_sems,
    )

    final1_k = pl.kernel(
        functools.partial(_iter1_body, nw, True),
        out_type=(
            jax.ShapeDtypeStruct((S,), f32),
            jax.ShapeDtypeStruct((Q, TCH), jnp.int32),
            jax.ShapeDtypeStruct((Q, TCH), f32),
        ),
        mesh=mesh,
        compiler_params=cparams,
        scratch_types=[
            pltpu.VMEM((S,), f32),
            pltpu.VMEM((2 * FCH,), jnp.int32),
            pltpu.VMEM((2 * TCH,), f32),
            pltpu.VMEM((spt,), f32),
            pltpu.VMEM((TCH,), jnp.int32),
            pltpu.VMEM((TCH,), f32),
        ] + dma_sems,
    )

    loop2_k = pl.kernel(
        functools.partial(_loop2_body, ns),
        out_type=jax.ShapeDtypeStruct((S,), f32),
        mesh=mesh,
        compiler_params=cparams,
        scratch_types=[
            pltpu.VMEM((S,), f32),
            pltpu.VMEM((2 * 4 * TCH,), jnp.int32),
            pltpu.VMEM((2 * 4 * TCH,), f32),
            pltpu.VMEM((S // ns,), f32),
            pltpu.VMEM_SHARED((S,), f32),
        ] + dma_sems,
    )

    twr = init_wr
    for _ in range(N_ITER - 1):
        twr = iter1_k(fT, tpR, twr)
    twr, gT, ohT = final1_k(fT, tpR, twr)

    gT4 = gT.reshape(Q // 4, 4 * TCH)
    tpR4 = tpR.reshape(Q // 4, 4 * TCH)
    wr2 = loop2_k(gT4, tpR4, init_wr)

    twr4 = twr.reshape(16, 16, 16, 16)
    v = twr4[1, 1, 0, 0]
    ub = 1.0 - (1.0 - v) / (v + 1e-9)

    mask_key = jax.random.key(42)
    mask = (jax.random.uniform(mask_key, twr4.shape) < MASK_STD).astype(f32)
    loss = jnp.sum(mask * twr4)

    opt_policy_dist = (ohT.reshape(Q, A, C).transpose(0, 2, 1)
                       .reshape(S, A).astype(policy.dtype))

    wr4 = wr2.reshape(16, 16, 16, 16)
    v2 = wr4[1, 1, 0, 0]
    lb = 1.0 - (1.0 - v2) / (v2 + 1e-9)

    return (ub, lb, opt_policy_dist, twr4, loss)
